# baseline (speedup 1.0000x reference)
import functools
import math

import numpy as np
import jax
import jax.numpy as jnp
from jax import lax
from jax.experimental import pallas as pl
from jax.experimental.pallas import tpu as pltpu
from jax.experimental.pallas import tpu_sc as plsc

D_MODEL = 64
_SCALE = 8.0
_L = 16


@jax.jit
def _run(xt, table, pe):
    S, NW, BB = xt.shape
    V, D = table.shape
    NB = 8
    LEAD = 6
    n_groups = S // NB

    mesh = plsc.VectorSubcoreMesh(core_axis_name="c", subcore_axis_name="s")

    @functools.partial(
        pl.kernel,
        out_type=jax.ShapeDtypeStruct((S, D // 8, NW, 8, BB), jnp.float32),
        mesh=mesh,
        scratch_types=[
            pltpu.VMEM((S, BB), jnp.int32),
        ]
        + [pltpu.VMEM((BB, D), jnp.float32)] * NB
        + [
            pltpu.VMEM((D // 8, 8, BB + 8), jnp.float32),
            pltpu.VMEM((D // 8, 8, BB + 8), jnp.float32),
            pltpu.VMEM((S, D), jnp.float32),
        ]
        + [pltpu.SemaphoreType.DMA] * (NB + 3),
        compiler_params=pltpu.CompilerParams(
            use_tc_tiling_on_sc=False, needs_layout_passes=False
        ),
    )
    def sc_kernel(xt_hbm, table_hbm, pe_hbm, out_hbm, idx_all, *rest):
        bufs = rest[:NB]
        t0, t1, pe_v, isem = rest[NB : NB + 4]
        tbufs = (t0, t1)
        gsem = rest[NB + 4 : 2 * NB + 4]
        osem = rest[2 * NB + 4 :]
        bt = lax.axis_index("s") * 2 + lax.axis_index("c")

        pltpu.sync_copy(pe_hbm, pe_v)
        pltpu.async_copy(xt_hbm.at[:, bt], idx_all, isem).wait()

        def gather_start(p, b):
            pltpu.async_copy(table_hbm.at[idx_all.at[p]], bufs[b], gsem[b])

        def gather_wait(p, b):
            pltpu.make_async_copy(
                table_hbm.at[idx_all.at[p]], bufs[b], gsem[b]
            ).wait()

        def out_start(p, b):
            pltpu.async_copy(
                tbufs[b % 2].at[:, :, pl.ds(0, BB)],
                out_hbm.at[p, :, bt],
                osem[b % 2],
            )

        def out_wait(b):
            pltpu.make_async_copy(
                tbufs[b % 2].at[:, :, pl.ds(0, BB)],
                out_hbm.at[0, :, bt],
                osem[b % 2],
            ).wait()

        for b in range(LEAD):
            gather_start(b, b)

        lane = lax.iota(jnp.int32, 16)
        dts = [(lane + 16 * j) // 8 for j in range(D // _L)]
        drs = [lane % 8 for _ in range(D // _L)]

        def group_body(g, carry):
            for b in range(NB):
                p = NB * g + b
                gather_wait(p, b)
                if b < 2:

                    @pl.when(g > 0)
                    def _():
                        out_wait(b)

                else:
                    out_wait(b)

                pe4 = [pe_v[p, pl.ds(16 * j, _L)] for j in range(D // _L)]

                def i_body(i, ic):
                    ci = jnp.full((_L,), i, jnp.int32)
                    for j in range(D // _L):
                        v = bufs[b][i, pl.ds(16 * j, _L)] * _SCALE + pe4[j]
                        plsc.store_scatter(
                            tbufs[b % 2], [dts[j], drs[j], ci], v
                        )
                    return ic

                lax.fori_loop(0, BB, i_body, 0, unroll=2)
                out_start(p, b)

                @pl.when(p + LEAD < S)
                def _():
                    gather_start(p + LEAD, (b + LEAD) % NB)

            return carry

        lax.fori_loop(0, n_groups, group_body, 0)
        out_wait(0)
        out_wait(1)

    return sc_kernel(xt, table, pe)


def kernel(x, table):
    B, S = x.shape
    V, D = table.shape
    NW = 32
    BB = B // NW

    pe = np.zeros((S, D_MODEL), dtype=np.float32)
    pos = np.arange(S, dtype=np.float32)[:, None]
    div_term = np.exp(
        np.arange(0, D_MODEL, 2, dtype=np.float32) * (-math.log(10000.0) / D_MODEL)
    )
    pe[:, 0::2] = np.sin(pos * div_term)
    pe[:, 1::2] = np.cos(pos * div_term)

    xt = x.T.reshape(S, NW, BB)
    out5 = _run(xt, table, jnp.asarray(pe))
    return out5.transpose(2, 4, 0, 1, 3).reshape(B, S, D)


# speedup vs baseline: 3.0453x; 3.0453x over previous
import functools
import math

import numpy as np
import jax
import jax.numpy as jnp
from jax import lax
from jax.experimental import pallas as pl
from jax.experimental.pallas import tpu as pltpu
from jax.experimental.pallas import tpu_sc as plsc

D_MODEL = 64
_SCALE = 8.0
_L = 16


@jax.jit
def _run(xt, table, pe):
    S, NW, BB = xt.shape
    V, D = table.shape
    NB = 8
    LEAD = 6
    n_groups = S // NB

    mesh = plsc.VectorSubcoreMesh(core_axis_name="c", subcore_axis_name="s")

    @functools.partial(
        pl.kernel,
        out_type=jax.ShapeDtypeStruct((S, D // 8, NW, 8, BB), jnp.float32),
        mesh=mesh,
        scratch_types=[
            pltpu.VMEM((S, BB), jnp.int32),
        ]
        + [pltpu.VMEM((BB, D), jnp.float32)] * NB
        + [
            pltpu.VMEM((D // 8, 8, BB + 8), jnp.float32),
            pltpu.VMEM((D // 8, 8, BB + 8), jnp.float32),
            pltpu.VMEM((S, D), jnp.float32),
        ]
        + [pltpu.SemaphoreType.DMA] * (NB + 3),
        compiler_params=pltpu.CompilerParams(
            use_tc_tiling_on_sc=False, needs_layout_passes=False
        ),
    )
    def sc_kernel(xt_hbm, table_hbm, pe_hbm, out_hbm, idx_all, *rest):
        bufs = rest[:NB]
        t0, t1, pe_v, isem = rest[NB : NB + 4]
        tbufs = (t0, t1)
        gsem = rest[NB + 4 : 2 * NB + 4]
        osem = rest[2 * NB + 4 :]
        bt = lax.axis_index("s") * 2 + lax.axis_index("c")

        pltpu.sync_copy(pe_hbm, pe_v)
        pltpu.async_copy(xt_hbm.at[:, bt], idx_all, isem).wait()

        def gather_start(p, b):
            pltpu.async_copy(table_hbm.at[idx_all.at[p]], bufs[b], gsem[b])

        def gather_wait(p, b):
            pltpu.make_async_copy(
                table_hbm.at[idx_all.at[p]], bufs[b], gsem[b]
            ).wait()

        def out_start(p, b):
            pltpu.async_copy(
                tbufs[b % 2].at[:, :, pl.ds(0, BB)],
                out_hbm.at[p, :, bt],
                osem[b % 2],
            )

        def out_wait(b):
            pltpu.make_async_copy(
                tbufs[b % 2].at[:, :, pl.ds(0, BB)],
                out_hbm.at[0, :, bt],
                osem[b % 2],
            ).wait()

        for b in range(LEAD):
            gather_start(b, b)

        lane = lax.iota(jnp.int32, 16)
        dts = [(lane + 16 * j) // 8 for j in range(D // _L)]
        drs = [lane % 8 for _ in range(D // _L)]

        def group_body(g, carry):
            for b in range(NB):
                p = NB * g + b
                gather_wait(p, b)
                if b < 2:

                    @pl.when(g > 0)
                    def _():
                        out_wait(b)

                else:
                    out_wait(b)

                pe4 = [pe_v[p, pl.ds(16 * j, _L)] for j in range(D // _L)]

                @plsc.parallel_loop(0, BB, unroll=8)
                def _(i):
                    ci = jnp.full((_L,), i, jnp.int32)
                    for j in range(D // _L):
                        v = bufs[b][i, pl.ds(16 * j, _L)] * _SCALE + pe4[j]
                        plsc.store_scatter(
                            tbufs[b % 2], [dts[j], drs[j], ci], v
                        )
                out_start(p, b)

                @pl.when(p + LEAD < S)
                def _():
                    gather_start(p + LEAD, (b + LEAD) % NB)

            return carry

        lax.fori_loop(0, n_groups, group_body, 0)
        out_wait(0)
        out_wait(1)

    return sc_kernel(xt, table, pe)


def kernel(x, table):
    B, S = x.shape
    V, D = table.shape
    NW = 32
    BB = B // NW

    pe = np.zeros((S, D_MODEL), dtype=np.float32)
    pos = np.arange(S, dtype=np.float32)[:, None]
    div_term = np.exp(
        np.arange(0, D_MODEL, 2, dtype=np.float32) * (-math.log(10000.0) / D_MODEL)
    )
    pe[:, 0::2] = np.sin(pos * div_term)
    pe[:, 1::2] = np.cos(pos * div_term)

    xt = x.T.reshape(S, NW, BB)
    out5 = _run(xt, table, jnp.asarray(pe))
    return out5.transpose(2, 4, 0, 1, 3).reshape(B, S, D)


# speedup vs baseline: 3.0613x; 1.0053x over previous
import functools
import math

import numpy as np
import jax
import jax.numpy as jnp
from jax import lax
from jax.experimental import pallas as pl
from jax.experimental.pallas import tpu as pltpu
from jax.experimental.pallas import tpu_sc as plsc

D_MODEL = 64
_SCALE = 8.0
_L = 16


@jax.jit
def _run(xt, table, pe):
    PT, NW, PR, BB = xt.shape
    S = PT * PR
    V, D = table.shape
    NB = 8
    LEAD = 6
    n_groups = S // NB

    mesh = plsc.VectorSubcoreMesh(core_axis_name="c", subcore_axis_name="s")

    @functools.partial(
        pl.kernel,
        out_type=jax.ShapeDtypeStruct((S, D // 8, NW, 8, BB), jnp.float32),
        mesh=mesh,
        scratch_types=[
            pltpu.VMEM((PT, PR, BB), jnp.int32),
        ]
        + [pltpu.VMEM((BB, D), jnp.float32)] * NB
        + [
            pltpu.VMEM((D // 8, 8, BB + 8), jnp.float32),
            pltpu.VMEM((D // 8, 8, BB + 8), jnp.float32),
            pltpu.VMEM((S, D), jnp.float32),
        ]
        + [pltpu.SemaphoreType.DMA] * (NB + 3),
        compiler_params=pltpu.CompilerParams(
            use_tc_tiling_on_sc=False, needs_layout_passes=False
        ),
    )
    def sc_kernel(xt_hbm, table_hbm, pe_hbm, out_hbm, idx_all, *rest):
        bufs = rest[:NB]
        t0, t1, pe_v, isem = rest[NB : NB + 4]
        tbufs = (t0, t1)
        gsem = rest[NB + 4 : 2 * NB + 4]
        osem = rest[2 * NB + 4 :]
        bt = lax.axis_index("s") * 2 + lax.axis_index("c")

        pltpu.sync_copy(pe_hbm, pe_v)
        pltpu.async_copy(xt_hbm.at[:, bt], idx_all, isem).wait()

        def gather_start(p, b):
            pltpu.async_copy(
                table_hbm.at[idx_all.at[p // PR, p % PR]], bufs[b], gsem[b]
            )

        def gather_wait(p, b):
            pltpu.make_async_copy(
                table_hbm.at[idx_all.at[p // PR, p % PR]], bufs[b], gsem[b]
            ).wait()

        def out_start(p, b):
            pltpu.async_copy(
                tbufs[b % 2].at[:, :, pl.ds(0, BB)],
                out_hbm.at[p, :, bt],
                osem[b % 2],
            )

        def out_wait(b):
            pltpu.make_async_copy(
                tbufs[b % 2].at[:, :, pl.ds(0, BB)],
                out_hbm.at[0, :, bt],
                osem[b % 2],
            ).wait()

        for b in range(LEAD):
            gather_start(b, b)

        lane = lax.iota(jnp.int32, 16)
        dts = [(lane + 16 * j) // 8 for j in range(D // _L)]
        drs = [lane % 8 for _ in range(D // _L)]

        def group_body(g, carry):
            for b in range(NB):
                p = NB * g + b
                gather_wait(p, b)
                if b < 2:

                    @pl.when(g > 0)
                    def _():
                        out_wait(b)

                else:
                    out_wait(b)

                pe4 = [pe_v[p, pl.ds(16 * j, _L)] for j in range(D // _L)]

                @plsc.parallel_loop(0, BB, unroll=8)
                def _(i):
                    ci = jnp.full((_L,), i, jnp.int32)
                    for j in range(D // _L):
                        v = bufs[b][i, pl.ds(16 * j, _L)] * _SCALE + pe4[j]
                        plsc.store_scatter(
                            tbufs[b % 2], [dts[j], drs[j], ci], v
                        )
                out_start(p, b)

                @pl.when(p + LEAD < S)
                def _():
                    gather_start(p + LEAD, (b + LEAD) % NB)

            return carry

        lax.fori_loop(0, n_groups, group_body, 0)
        out_wait(0)
        out_wait(1)

    return sc_kernel(xt, table, pe)


def kernel(x, table):
    B, S = x.shape
    V, D = table.shape
    NW = 32
    BB = B // NW

    pe = np.zeros((S, D_MODEL), dtype=np.float32)
    pos = np.arange(S, dtype=np.float32)[:, None]
    div_term = np.exp(
        np.arange(0, D_MODEL, 2, dtype=np.float32) * (-math.log(10000.0) / D_MODEL)
    )
    pe[:, 0::2] = np.sin(pos * div_term)
    pe[:, 1::2] = np.cos(pos * div_term)

    xq = x.T.reshape(S // 8, 8, NW, BB).transpose(0, 2, 1, 3)
    out5 = _run(xq, table, jnp.asarray(pe))
    return out5.transpose(2, 4, 0, 1, 3).reshape(B, S, D)


# speedup vs baseline: 3.0649x; 1.0012x over previous
import functools
import math

import numpy as np
import jax
import jax.numpy as jnp
from jax import lax
from jax.experimental import pallas as pl
from jax.experimental.pallas import tpu as pltpu
from jax.experimental.pallas import tpu_sc as plsc

D_MODEL = 64
_SCALE = 8.0
_L = 16


@jax.jit
def _run(xt, table, pe):
    PT, NW, PR, BB = xt.shape
    S = PT * PR
    V, D = table.shape
    NB = 8
    LEAD = 6
    n_groups = S // NB

    mesh = plsc.VectorSubcoreMesh(core_axis_name="c", subcore_axis_name="s")

    @functools.partial(
        pl.kernel,
        out_type=jax.ShapeDtypeStruct((S, D // 8, NW, 8, BB), jnp.float32),
        mesh=mesh,
        scratch_types=[
            pltpu.VMEM((PT, PR, BB), jnp.int32),
        ]
        + [pltpu.VMEM((BB, D), jnp.float32)] * NB
        + [
            pltpu.VMEM((D // 8, 8, BB + 8), jnp.float32),
            pltpu.VMEM((D // 8, 8, BB + 8), jnp.float32),
            pltpu.VMEM((S, D), jnp.float32),
        ]
        + [pltpu.SemaphoreType.DMA] * (NB + 3),
        compiler_params=pltpu.CompilerParams(
            use_tc_tiling_on_sc=False, needs_layout_passes=False
        ),
    )
    def sc_kernel(xt_hbm, table_hbm, pe_hbm, out_hbm, idx_all, *rest):
        bufs = rest[:NB]
        t0, t1, pe_v, isem = rest[NB : NB + 4]
        tbufs = (t0, t1)
        gsem = rest[NB + 4 : 2 * NB + 4]
        osem = rest[2 * NB + 4 :]
        bt = lax.axis_index("s") * 2 + lax.axis_index("c")

        pltpu.async_copy(
            xt_hbm.at[pl.ds(0, 1), bt], idx_all.at[pl.ds(0, 1)], isem
        ).wait()

        def gather_start(p, b):
            pltpu.async_copy(
                table_hbm.at[idx_all.at[p // PR, p % PR]], bufs[b], gsem[b]
            )

        def gather_wait(p, b):
            pltpu.make_async_copy(
                table_hbm.at[idx_all.at[p // PR, p % PR]], bufs[b], gsem[b]
            ).wait()

        def out_start(p, b):
            pltpu.async_copy(
                tbufs[b % 2].at[:, :, pl.ds(0, BB)],
                out_hbm.at[p, :, bt],
                osem[b % 2],
            )

        def out_wait(b):
            pltpu.make_async_copy(
                tbufs[b % 2].at[:, :, pl.ds(0, BB)],
                out_hbm.at[0, :, bt],
                osem[b % 2],
            ).wait()

        for b in range(LEAD):
            gather_start(b, b)
        pltpu.async_copy(
            xt_hbm.at[pl.ds(1, PT - 1), bt], idx_all.at[pl.ds(1, PT - 1)], isem
        )
        pltpu.sync_copy(pe_hbm, pe_v)
        pltpu.make_async_copy(
            xt_hbm.at[pl.ds(1, PT - 1), bt], idx_all.at[pl.ds(1, PT - 1)], isem
        ).wait()

        lane = lax.iota(jnp.int32, 16)
        dts = [(lane + 16 * j) // 8 for j in range(D // _L)]
        drs = [lane % 8 for _ in range(D // _L)]

        def group_body(g, carry):
            for b in range(NB):
                p = NB * g + b
                gather_wait(p, b)
                if b < 2:

                    @pl.when(g > 0)
                    def _():
                        out_wait(b)

                else:
                    out_wait(b)

                pe4 = [pe_v[p, pl.ds(16 * j, _L)] for j in range(D // _L)]

                @plsc.parallel_loop(0, BB, unroll=8)
                def _(i):
                    ci = jnp.full((_L,), i, jnp.int32)
                    for j in range(D // _L):
                        v = bufs[b][i, pl.ds(16 * j, _L)] * _SCALE + pe4[j]
                        plsc.store_scatter(
                            tbufs[b % 2], [dts[j], drs[j], ci], v
                        )
                out_start(p, b)

                @pl.when(p + LEAD < S)
                def _():
                    gather_start(p + LEAD, (b + LEAD) % NB)

            return carry

        lax.fori_loop(0, n_groups, group_body, 0)
        out_wait(0)
        out_wait(1)

    return sc_kernel(xt, table, pe)


def kernel(x, table):
    B, S = x.shape
    V, D = table.shape
    NW = 32
    BB = B // NW

    pe = np.zeros((S, D_MODEL), dtype=np.float32)
    pos = np.arange(S, dtype=np.float32)[:, None]
    div_term = np.exp(
        np.arange(0, D_MODEL, 2, dtype=np.float32) * (-math.log(10000.0) / D_MODEL)
    )
    pe[:, 0::2] = np.sin(pos * div_term)
    pe[:, 1::2] = np.cos(pos * div_term)

    xq = x.T.reshape(S // 8, 8, NW, BB).transpose(0, 2, 1, 3)
    out5 = _run(xq, table, jnp.asarray(pe))
    return out5.transpose(2, 4, 0, 1, 3).reshape(B, S, D)
